# baseline (speedup 1.0000x reference)
import functools

import jax
import jax.numpy as jnp
from jax import lax
from jax.experimental import pallas as pl
from jax.experimental.pallas import tpu as pltpu
from jax.experimental.pallas import tpu_sc as plsc

_N = 10000
_NP = 10240
_E = 160000
_K = 80
_NC = 2
_NS = 16
_BN = 2000


def _segsum_sc(gidx, dst, table):
    ept = _E // _NS
    nch = ept // _K
    zrows = _NP // _NS
    mesh = plsc.VectorSubcoreMesh(core_axis_name="c", subcore_axis_name="s",
                                  num_cores=_NC, num_subcores=_NS)

    @functools.partial(
        pl.kernel,
        out_type=jax.ShapeDtypeStruct((2 * _NP, 128), jnp.float32),
        mesh=mesh,
        scratch_types=[
            pltpu.VMEM((_K,), jnp.int32),
            pltpu.VMEM((_K,), jnp.int32),
            pltpu.VMEM((_K, 128), jnp.float32),
            pltpu.VMEM_SHARED((_NP, 128), jnp.float32),
            pltpu.SemaphoreType.DMA,
        ],
    )
    def k(gidx_hbm, dst_hbm, table_hbm, out_hbm, idx_v, dst_v, rows_v, acc_sh, sem):
        c = lax.axis_index("c")
        s = lax.axis_index("s")

        @pl.loop(0, _K)
        def _(r):
            for j in range(8):
                rows_v[r, pl.ds(j * 16, 16)] = jnp.zeros((16,), jnp.float32)

        zb = s * zrows

        @pl.loop(0, zrows // _K)
        def _(t):
            pltpu.sync_copy(rows_v, acc_sh.at[pl.ds(zb + t * _K, _K)])

        plsc.subcore_barrier()

        @pl.loop(0, nch)
        def _(g):
            off = s * ept + g * _K
            pltpu.sync_copy(gidx_hbm.at[pl.ds(c * _E + off, _K)], idx_v)
            pltpu.sync_copy(dst_hbm.at[pl.ds(off, _K)], dst_v)
            pltpu.async_copy(table_hbm.at[idx_v], rows_v, sem).wait()
            pltpu.sync_copy(rows_v, acc_sh.at[dst_v], add=True)

        plsc.subcore_barrier()
        pltpu.sync_copy(acc_sh.at[pl.ds(zb, zrows)],
                        out_hbm.at[pl.ds(c * _NP + zb, zrows)])

    return k(gidx, dst, table).reshape(2, _NP, 128)


def _combine_tc(terms, dout, leaky):
    args = []
    in_specs = []
    for halves, scale, ind, w, b2d in terms:
        for h in halves:
            args.append(h)
            in_specs.append(pl.BlockSpec((2, _BN, 128), lambda i: (0, i, 0)))
        args += [scale, ind, w, b2d]
        in_specs += [
            pl.BlockSpec((_BN, 1), lambda i: (i, 0)),
            pl.BlockSpec((_BN, 1), lambda i: (i, 0)),
            pl.BlockSpec(w.shape, lambda i: (0, 0)),
            pl.BlockSpec((1, dout), lambda i: (0, 0)),
        ]
    counts = [len(t[0]) for t in terms]

    def body(*refs):
        out_ref = refs[-1]
        acc = jnp.zeros((_BN, dout), jnp.float32)
        p = 0
        for cnt in counts:
            hrefs = refs[p:p + cnt]
            s_ref, i_ref, w_ref, b_ref = refs[p + cnt:p + cnt + 4]
            p += cnt + 4
            sc = s_ref[...]
            w = w_ref[...]
            for hi in range(cnt):
                x = hrefs[hi][...]
                for cc in range(2):
                    kb = hi * 2 + cc
                    acc += jnp.dot(x[cc] * sc, w[kb * 128:(kb + 1) * 128, :],
                                   preferred_element_type=jnp.float32)
            acc += i_ref[...] * b_ref[...]
        if leaky:
            acc = jnp.where(acc >= 0, acc, 0.01 * acc)
        out_ref[...] = acc

    return pl.pallas_call(
        body,
        grid=(_N // _BN,),
        in_specs=in_specs,
        out_specs=pl.BlockSpec((_BN, dout), lambda i: (i, 0)),
        out_shape=jax.ShapeDtypeStruct((_N, dout), jnp.float32),
    )(*args)


def _linear_tc(x, w, b2d):
    kdim = x.shape[1]
    dout = w.shape[1]

    def body(x_ref, w_ref, b_ref, out_ref):
        out_ref[...] = jnp.dot(x_ref[...], w_ref[...],
                               preferred_element_type=jnp.float32) + b_ref[...]

    return pl.pallas_call(
        body,
        grid=(_N // _BN,),
        in_specs=[
            pl.BlockSpec((_BN, kdim), lambda i: (i, 0)),
            pl.BlockSpec((kdim, dout), lambda i: (0, 0)),
            pl.BlockSpec((1, dout), lambda i: (0, 0)),
        ],
        out_specs=pl.BlockSpec((_BN, dout), lambda i: (i, 0)),
        out_shape=jax.ShapeDtypeStruct((_N, dout), jnp.float32),
    )(x, w, b2d)


def _final_tc(sff, rff, srb, rrb):
    def body(a_ref, ra_ref, b_ref, rb_ref, out_ref):
        a = a_ref[...]
        b = b_ref[...]
        ra = ra_ref[...]
        rb = rb_ref[...]
        out_ref[...] = jnp.concatenate(
            [a[0] * ra + b[0] * rb, a[1] * ra + b[1] * rb], axis=1)

    return pl.pallas_call(
        body,
        grid=(_N // _BN,),
        in_specs=[
            pl.BlockSpec((2, _BN, 128), lambda i: (0, i, 0)),
            pl.BlockSpec((_BN, 1), lambda i: (i, 0)),
            pl.BlockSpec((2, _BN, 128), lambda i: (0, i, 0)),
            pl.BlockSpec((_BN, 1), lambda i: (i, 0)),
        ],
        out_specs=pl.BlockSpec((_BN, 256), lambda i: (i, 0)),
        out_shape=jax.ShapeDtypeStruct((_N, 256), jnp.float32),
    )(sff, rff, srb, rrb)


def _gidx(src, m, o):
    base = m * src + o
    return jnp.concatenate([base, base + 1]).astype(jnp.int32)


def kernel(feat_user, feat_item, edge_ff, edge_rt, edge_rb,
           W1_ff, b1_ff, W1_rt, b1_rt, W1_rb, b1_rb,
           W2_ff, b2_ff, W2_rt, b2_rt, W2_rb, b2_rb,
           W3_ff, b3_ff, W3_rt, b3_rt, W3_rb, b3_rb):
    src_ff = edge_ff[0].astype(jnp.int32)
    dst_ff = edge_ff[1].astype(jnp.int32)
    src_rt = edge_rt[0].astype(jnp.int32)
    dst_rt = edge_rt[1].astype(jnp.int32)
    src_rb = edge_rb[0].astype(jnp.int32)
    dst_rb = edge_rb[1].astype(jnp.int32)

    ones_tab = jnp.ones((8, 128), jnp.float32)
    zidx = jnp.zeros((2 * _E,), jnp.int32)
    d_ff = _segsum_sc(zidx, dst_ff, ones_tab)[0, :_N, 0]
    d_rt = _segsum_sc(zidx, dst_rt, ones_tab)[0, :_N, 0]
    d_rb = _segsum_sc(zidx, dst_rb, ones_tab)[0, :_N, 0]

    def prep(d):
        r = (1.0 / jnp.maximum(d, 1.0)).reshape(_N, 1)
        ind = (d > 0).astype(jnp.float32).reshape(_N, 1)
        return r, ind

    r_ff, i_ff = prep(d_ff)
    r_rt, i_rt = prep(d_rt)
    r_rb, i_rb = prep(d_rb)

    b2 = lambda b: b.reshape(1, -1)

    g2_ff = _gidx(src_ff, 2, 0)
    g2_rt = _gidx(src_rt, 2, 0)
    g2_rb = _gidx(src_rb, 2, 0)
    tu = feat_user.reshape(2 * _N, 128)
    ti = feat_item.reshape(2 * _N, 128)
    s_ff1 = _segsum_sc(g2_ff, dst_ff, tu)
    s_rt1 = _segsum_sc(g2_rt, dst_rt, tu)
    s_rb1 = _segsum_sc(g2_rb, dst_rb, ti)
    hu1 = _combine_tc([([s_ff1], r_ff, i_ff, W1_ff, b2(b1_ff)),
                       ([s_rb1], r_rb, i_rb, W1_rb, b2(b1_rb))], 512, True)
    hi1 = _combine_tc([([s_rt1], r_rt, i_rt, W1_rt, b2(b1_rt))], 512, True)

    tu2 = hu1.reshape(4 * _N, 128)
    ti2 = hi1.reshape(4 * _N, 128)
    g4a_ff = _gidx(src_ff, 4, 0)
    g4b_ff = _gidx(src_ff, 4, 2)
    g4a_rt = _gidx(src_rt, 4, 0)
    g4b_rt = _gidx(src_rt, 4, 2)
    g4a_rb = _gidx(src_rb, 4, 0)
    g4b_rb = _gidx(src_rb, 4, 2)
    s_ff2a = _segsum_sc(g4a_ff, dst_ff, tu2)
    s_ff2b = _segsum_sc(g4b_ff, dst_ff, tu2)
    s_rt2a = _segsum_sc(g4a_rt, dst_rt, tu2)
    s_rt2b = _segsum_sc(g4b_rt, dst_rt, tu2)
    s_rb2a = _segsum_sc(g4a_rb, dst_rb, ti2)
    s_rb2b = _segsum_sc(g4b_rb, dst_rb, ti2)
    hu2 = _combine_tc([([s_ff2a, s_ff2b], r_ff, i_ff, W2_ff, b2(b2_ff)),
                       ([s_rb2a, s_rb2b], r_rb, i_rb, W2_rb, b2(b2_rb))], 512, True)
    hi2 = _combine_tc([([s_rt2a, s_rt2b], r_rt, i_rt, W2_rt, b2(b2_rt))], 512, True)

    p_ff = _linear_tc(hu2, W3_ff, b2(b3_ff))
    p_rb = _linear_tc(hi2, W3_rb, b2(b3_rb))
    s_ff3 = _segsum_sc(g2_ff, dst_ff, p_ff.reshape(2 * _N, 128))
    s_rb3 = _segsum_sc(g2_rb, dst_rb, p_rb.reshape(2 * _N, 128))
    return _final_tc(s_ff3, r_ff, s_rb3, r_rb)


# speedup vs baseline: 9.0701x; 9.0701x over previous
import functools

import jax
import jax.numpy as jnp
from jax import lax
from jax.experimental import pallas as pl
from jax.experimental.pallas import tpu as pltpu
from jax.experimental.pallas import tpu_sc as plsc

_N = 10000
_NP = 10240
_E = 160000
_K = 80
_NC = 2
_NS = 16
_BN = 2000


def _segsum_sc(gidx, dst, table):
    ept = _E // _NS
    nch = ept // _K
    zrows = _NP // _NS
    mesh = plsc.VectorSubcoreMesh(core_axis_name="c", subcore_axis_name="s",
                                  num_cores=_NC, num_subcores=_NS)

    @functools.partial(
        pl.kernel,
        out_type=jax.ShapeDtypeStruct((2 * _NP, 128), jnp.float32),
        mesh=mesh,
        scratch_types=[
            pltpu.VMEM((_K,), jnp.int32),
            pltpu.VMEM((_K,), jnp.int32),
            pltpu.VMEM((_K, 128), jnp.float32),
            pltpu.VMEM_SHARED((_NP, 128), jnp.float32),
            pltpu.SemaphoreType.DMA,
        ],
    )
    def k(gidx_hbm, dst_hbm, table_hbm, out_hbm, idx_v, dst_v, rows_v, acc_sh, sem):
        c = lax.axis_index("c")
        s = lax.axis_index("s")

        @pl.loop(0, _K)
        def _(r):
            for j in range(8):
                rows_v[r, pl.ds(j * 16, 16)] = jnp.zeros((16,), jnp.float32)

        zb = s * zrows

        @pl.loop(0, zrows // _K)
        def _(t):
            pltpu.sync_copy(rows_v, acc_sh.at[pl.ds(zb + t * _K, _K)])

        plsc.subcore_barrier()

        @pl.loop(0, nch)
        def _(g):
            off = s * ept + g * _K
            pltpu.sync_copy(gidx_hbm.at[pl.ds(c * _E + off, _K)], idx_v)
            pltpu.sync_copy(dst_hbm.at[pl.ds(off, _K)], dst_v)
            pltpu.async_copy(table_hbm.at[idx_v], rows_v, sem).wait()
            pltpu.sync_copy(rows_v, acc_sh.at[dst_v], add=True)

        plsc.subcore_barrier()
        pltpu.sync_copy(acc_sh.at[pl.ds(zb, zrows)],
                        out_hbm.at[pl.ds(c * _NP + zb, zrows)])

    return k(gidx, dst, table).reshape(2, _NP, 128)


def _combine_tc(terms, dout, leaky):
    args = []
    in_specs = []
    for halves, scale, ind, w, b2d in terms:
        for h in halves:
            args.append(h)
            in_specs.append(pl.BlockSpec((2, _BN, 128), lambda i: (0, i, 0)))
        args += [scale, ind, w, b2d]
        in_specs += [
            pl.BlockSpec((_BN, 1), lambda i: (i, 0)),
            pl.BlockSpec((_BN, 1), lambda i: (i, 0)),
            pl.BlockSpec(w.shape, lambda i: (0, 0)),
            pl.BlockSpec((1, dout), lambda i: (0, 0)),
        ]
    counts = [len(t[0]) for t in terms]

    def body(*refs):
        out_ref = refs[-1]
        acc = jnp.zeros((_BN, dout), jnp.float32)
        p = 0
        for cnt in counts:
            hrefs = refs[p:p + cnt]
            s_ref, i_ref, w_ref, b_ref = refs[p + cnt:p + cnt + 4]
            p += cnt + 4
            sc = s_ref[...]
            w = w_ref[...]
            for hi in range(cnt):
                x = hrefs[hi][...]
                for cc in range(2):
                    kb = hi * 2 + cc
                    acc += jnp.dot(x[cc] * sc, w[kb * 128:(kb + 1) * 128, :],
                                   preferred_element_type=jnp.float32)
            acc += i_ref[...] * b_ref[...]
        if leaky:
            acc = jnp.where(acc >= 0, acc, 0.01 * acc)
        out_ref[...] = acc

    return pl.pallas_call(
        body,
        grid=(_N // _BN,),
        in_specs=in_specs,
        out_specs=pl.BlockSpec((_BN, dout), lambda i: (i, 0)),
        out_shape=jax.ShapeDtypeStruct((_N, dout), jnp.float32),
    )(*args)


def _linear_tc(x, w, b2d):
    kdim = x.shape[1]
    dout = w.shape[1]

    def body(x_ref, w_ref, b_ref, out_ref):
        out_ref[...] = jnp.dot(x_ref[...], w_ref[...],
                               preferred_element_type=jnp.float32) + b_ref[...]

    return pl.pallas_call(
        body,
        grid=(_N // _BN,),
        in_specs=[
            pl.BlockSpec((_BN, kdim), lambda i: (i, 0)),
            pl.BlockSpec((kdim, dout), lambda i: (0, 0)),
            pl.BlockSpec((1, dout), lambda i: (0, 0)),
        ],
        out_specs=pl.BlockSpec((_BN, dout), lambda i: (i, 0)),
        out_shape=jax.ShapeDtypeStruct((_N, dout), jnp.float32),
    )(x, w, b2d)


def _final_tc(sff, rff, srb, rrb):
    def body(a_ref, ra_ref, b_ref, rb_ref, out_ref):
        a = a_ref[...]
        b = b_ref[...]
        ra = ra_ref[...]
        rb = rb_ref[...]
        out_ref[...] = jnp.concatenate(
            [a[0] * ra + b[0] * rb, a[1] * ra + b[1] * rb], axis=1)

    return pl.pallas_call(
        body,
        grid=(_N // _BN,),
        in_specs=[
            pl.BlockSpec((2, _BN, 128), lambda i: (0, i, 0)),
            pl.BlockSpec((_BN, 1), lambda i: (i, 0)),
            pl.BlockSpec((2, _BN, 128), lambda i: (0, i, 0)),
            pl.BlockSpec((_BN, 1), lambda i: (i, 0)),
        ],
        out_specs=pl.BlockSpec((_BN, 256), lambda i: (i, 0)),
        out_shape=jax.ShapeDtypeStruct((_N, 256), jnp.float32),
    )(sff, rff, srb, rrb)


def _gidx(src, m, o):
    base = m * src + o
    return jnp.concatenate([base, base + 1]).astype(jnp.int32)


def kernel(feat_user, feat_item, edge_ff, edge_rt, edge_rb,
           W1_ff, b1_ff, W1_rt, b1_rt, W1_rb, b1_rb,
           W2_ff, b2_ff, W2_rt, b2_rt, W2_rb, b2_rb,
           W3_ff, b3_ff, W3_rt, b3_rt, W3_rb, b3_rb):
    src_ff = edge_ff[0].astype(jnp.int32)
    dst_ff = edge_ff[1].astype(jnp.int32)
    src_rt = edge_rt[0].astype(jnp.int32)
    dst_rt = edge_rt[1].astype(jnp.int32)
    src_rb = edge_rb[0].astype(jnp.int32)
    dst_rb = edge_rb[1].astype(jnp.int32)

    g2_ff = _gidx(src_ff, 2, 0)
    g2_rt = _gidx(src_rt, 2, 0)
    g2_rb = _gidx(src_rb, 2, 0)
    ones_tab = jnp.ones((2 * _N, 128), jnp.float32)
    d_ff = _segsum_sc(g2_ff, dst_ff, ones_tab)[0, :_N, 0]
    d_rt = _segsum_sc(g2_rt, dst_rt, ones_tab)[0, :_N, 0]
    d_rb = _segsum_sc(g2_rb, dst_rb, ones_tab)[0, :_N, 0]

    def prep(d):
        r = (1.0 / jnp.maximum(d, 1.0)).reshape(_N, 1)
        ind = (d > 0).astype(jnp.float32).reshape(_N, 1)
        return r, ind

    r_ff, i_ff = prep(d_ff)
    r_rt, i_rt = prep(d_rt)
    r_rb, i_rb = prep(d_rb)

    b2 = lambda b: b.reshape(1, -1)

    tu = feat_user.reshape(2 * _N, 128)
    ti = feat_item.reshape(2 * _N, 128)
    s_ff1 = _segsum_sc(g2_ff, dst_ff, tu)
    s_rt1 = _segsum_sc(g2_rt, dst_rt, tu)
    s_rb1 = _segsum_sc(g2_rb, dst_rb, ti)
    hu1 = _combine_tc([([s_ff1], r_ff, i_ff, W1_ff, b2(b1_ff)),
                       ([s_rb1], r_rb, i_rb, W1_rb, b2(b1_rb))], 512, True)
    hi1 = _combine_tc([([s_rt1], r_rt, i_rt, W1_rt, b2(b1_rt))], 512, True)

    tu2 = hu1.reshape(4 * _N, 128)
    ti2 = hi1.reshape(4 * _N, 128)
    g4a_ff = _gidx(src_ff, 4, 0)
    g4b_ff = _gidx(src_ff, 4, 2)
    g4a_rt = _gidx(src_rt, 4, 0)
    g4b_rt = _gidx(src_rt, 4, 2)
    g4a_rb = _gidx(src_rb, 4, 0)
    g4b_rb = _gidx(src_rb, 4, 2)
    s_ff2a = _segsum_sc(g4a_ff, dst_ff, tu2)
    s_ff2b = _segsum_sc(g4b_ff, dst_ff, tu2)
    s_rt2a = _segsum_sc(g4a_rt, dst_rt, tu2)
    s_rt2b = _segsum_sc(g4b_rt, dst_rt, tu2)
    s_rb2a = _segsum_sc(g4a_rb, dst_rb, ti2)
    s_rb2b = _segsum_sc(g4b_rb, dst_rb, ti2)
    hu2 = _combine_tc([([s_ff2a, s_ff2b], r_ff, i_ff, W2_ff, b2(b2_ff)),
                       ([s_rb2a, s_rb2b], r_rb, i_rb, W2_rb, b2(b2_rb))], 512, True)
    hi2 = _combine_tc([([s_rt2a, s_rt2b], r_rt, i_rt, W2_rt, b2(b2_rt))], 512, True)

    p_ff = _linear_tc(hu2, W3_ff, b2(b3_ff))
    p_rb = _linear_tc(hi2, W3_rb, b2(b3_rb))
    s_ff3 = _segsum_sc(g2_ff, dst_ff, p_ff.reshape(2 * _N, 128))
    s_rb3 = _segsum_sc(g2_rb, dst_rb, p_rb.reshape(2 * _N, 128))
    return _final_tc(s_ff3, r_ff, s_rb3, r_rb)


# speedup vs baseline: 18.4735x; 2.0368x over previous
import functools

import jax
import jax.numpy as jnp
from jax import lax
from jax.experimental import pallas as pl
from jax.experimental.pallas import tpu as pltpu
from jax.experimental.pallas import tpu_sc as plsc

_N = 10000
_NP = 10240
_E = 160000
_K = 80
_NC = 2
_NS = 16
_BN = 2000


_NCH = (_E // _NS) // _K
_NCHP = 128


def _pad_chunks(a, lead):
    a3 = a.reshape(lead, _NCH, _K)
    return jnp.pad(a3, ((0, 0), (0, _NCHP - _NCH), (0, 0))).reshape(-1, _K)


def _segsum_sc(gidx, dstp, table):
    zrows = _NP // _NS
    mesh = plsc.VectorSubcoreMesh(core_axis_name="c", subcore_axis_name="s",
                                  num_cores=_NC, num_subcores=_NS)

    @functools.partial(
        pl.kernel,
        out_type=jax.ShapeDtypeStruct((2 * _NP, 128), jnp.float32),
        mesh=mesh,
        scratch_types=[
            pltpu.VMEM((4 * _K,), jnp.int32),
            pltpu.VMEM((_NCHP, _K), jnp.int32),
            pltpu.VMEM((2, _K, 128), jnp.float32),
            pltpu.VMEM_SHARED((_NP, 128), jnp.float32),
            pltpu.SemaphoreType.DMA,
            pltpu.SemaphoreType.DMA,
        ],
    )
    def k(gidx_hbm, dst_hbm, table_hbm, out_hbm, idxg_v, dst_v, rows_v,
          acc_sh, gsem, ssem):
        c = lax.axis_index("c")
        s = lax.axis_index("s")
        ebase = c * _E + s * (_E // _NS)

        pltpu.sync_copy(dst_hbm.at[pl.ds(s * _NCHP, _NCHP)], dst_v)

        @pl.loop(0, _K)
        def _(r):
            for j in range(8):
                rows_v[0, r, pl.ds(j * 16, 16)] = jnp.zeros((16,), jnp.float32)

        zb = s * zrows

        @pl.loop(0, zrows // _K)
        def _(t):
            pltpu.sync_copy(rows_v.at[0], acc_sh.at[pl.ds(zb + t * _K, _K)])

        plsc.subcore_barrier()

        def gather(i, slot):
            return pltpu.async_copy(
                table_hbm.at[idxg_v.at[pl.ds(i * _K, _K)]],
                rows_v.at[slot], gsem)

        def scatter(g, slot):
            return pltpu.async_copy(rows_v.at[slot],
                                    acc_sh.at[dst_v.at[g]], ssem, add=True)

        @pl.loop(0, _NCH // 4)
        def _(j):
            base = 4 * j
            pltpu.sync_copy(gidx_hbm.at[pl.ds(ebase + base * _K, 4 * _K)],
                            idxg_v)
            g0 = gather(0, 0)
            g1 = gather(1, 1)
            g0.wait()
            s0 = scatter(base + 0, 0)
            g1.wait()
            s1 = scatter(base + 1, 1)
            s0.wait()
            g2 = gather(2, 0)
            s1.wait()
            g3 = gather(3, 1)
            g2.wait()
            s2 = scatter(base + 2, 0)
            g3.wait()
            s3 = scatter(base + 3, 1)
            s2.wait()
            s3.wait()

        pltpu.sync_copy(gidx_hbm.at[pl.ds(ebase + (_NCH - 1) * _K, _K)],
                        idxg_v.at[pl.ds(0, _K)])
        gather(0, 0).wait()
        scatter(_NCH - 1, 0).wait()

        plsc.subcore_barrier()
        pltpu.sync_copy(acc_sh.at[pl.ds(zb, zrows)],
                        out_hbm.at[pl.ds(c * _NP + zb, zrows)])

    return k(gidx, dstp, table).reshape(2, _NP, 128)


def _combine_tc(terms, dout, leaky):
    args = []
    in_specs = []
    for halves, scale, ind, w, b2d in terms:
        for h in halves:
            args.append(h)
            in_specs.append(pl.BlockSpec((2, _BN, 128), lambda i: (0, i, 0)))
        args += [scale, ind, w, b2d]
        in_specs += [
            pl.BlockSpec((_BN, 1), lambda i: (i, 0)),
            pl.BlockSpec((_BN, 1), lambda i: (i, 0)),
            pl.BlockSpec(w.shape, lambda i: (0, 0)),
            pl.BlockSpec((1, dout), lambda i: (0, 0)),
        ]
    counts = [len(t[0]) for t in terms]

    def body(*refs):
        out_ref = refs[-1]
        acc = jnp.zeros((_BN, dout), jnp.float32)
        p = 0
        for cnt in counts:
            hrefs = refs[p:p + cnt]
            s_ref, i_ref, w_ref, b_ref = refs[p + cnt:p + cnt + 4]
            p += cnt + 4
            sc = s_ref[...]
            w = w_ref[...]
            for hi in range(cnt):
                x = hrefs[hi][...]
                for cc in range(2):
                    kb = hi * 2 + cc
                    acc += jnp.dot(x[cc] * sc, w[kb * 128:(kb + 1) * 128, :],
                                   preferred_element_type=jnp.float32)
            acc += i_ref[...] * b_ref[...]
        if leaky:
            acc = jnp.where(acc >= 0, acc, 0.01 * acc)
        out_ref[...] = acc

    return pl.pallas_call(
        body,
        grid=(_N // _BN,),
        in_specs=in_specs,
        out_specs=pl.BlockSpec((_BN, dout), lambda i: (i, 0)),
        out_shape=jax.ShapeDtypeStruct((_N, dout), jnp.float32),
    )(*args)


def _linear_tc(x, w, b2d):
    kdim = x.shape[1]
    dout = w.shape[1]

    def body(x_ref, w_ref, b_ref, out_ref):
        out_ref[...] = jnp.dot(x_ref[...], w_ref[...],
                               preferred_element_type=jnp.float32) + b_ref[...]

    return pl.pallas_call(
        body,
        grid=(_N // _BN,),
        in_specs=[
            pl.BlockSpec((_BN, kdim), lambda i: (i, 0)),
            pl.BlockSpec((kdim, dout), lambda i: (0, 0)),
            pl.BlockSpec((1, dout), lambda i: (0, 0)),
        ],
        out_specs=pl.BlockSpec((_BN, dout), lambda i: (i, 0)),
        out_shape=jax.ShapeDtypeStruct((_N, dout), jnp.float32),
    )(x, w, b2d)


def _final_tc(sff, rff, srb, rrb):
    def body(a_ref, ra_ref, b_ref, rb_ref, out_ref):
        a = a_ref[...]
        b = b_ref[...]
        ra = ra_ref[...]
        rb = rb_ref[...]
        out_ref[...] = jnp.concatenate(
            [a[0] * ra + b[0] * rb, a[1] * ra + b[1] * rb], axis=1)

    return pl.pallas_call(
        body,
        grid=(_N // _BN,),
        in_specs=[
            pl.BlockSpec((2, _BN, 128), lambda i: (0, i, 0)),
            pl.BlockSpec((_BN, 1), lambda i: (i, 0)),
            pl.BlockSpec((2, _BN, 128), lambda i: (0, i, 0)),
            pl.BlockSpec((_BN, 1), lambda i: (i, 0)),
        ],
        out_specs=pl.BlockSpec((_BN, 256), lambda i: (i, 0)),
        out_shape=jax.ShapeDtypeStruct((_N, 256), jnp.float32),
    )(sff, rff, srb, rrb)


def _gidx(src, m, o):
    base = m * src + o
    return jnp.concatenate([base, base + 1]).astype(jnp.int32)


def kernel(feat_user, feat_item, edge_ff, edge_rt, edge_rb,
           W1_ff, b1_ff, W1_rt, b1_rt, W1_rb, b1_rb,
           W2_ff, b2_ff, W2_rt, b2_rt, W2_rb, b2_rb,
           W3_ff, b3_ff, W3_rt, b3_rt, W3_rb, b3_rb):
    src_ff = edge_ff[0].astype(jnp.int32)
    dst_ff = edge_ff[1].astype(jnp.int32)
    src_rt = edge_rt[0].astype(jnp.int32)
    dst_rt = edge_rt[1].astype(jnp.int32)
    src_rb = edge_rb[0].astype(jnp.int32)
    dst_rb = edge_rb[1].astype(jnp.int32)

    dstp_ff = _pad_chunks(dst_ff, _NS)
    dstp_rt = _pad_chunks(dst_rt, _NS)
    dstp_rb = _pad_chunks(dst_rb, _NS)
    g2_ff = _gidx(src_ff, 2, 0)
    g2_rt = _gidx(src_rt, 2, 0)
    g2_rb = _gidx(src_rb, 2, 0)

    ones_tab = jnp.ones((2 * _N, 128), jnp.float32)
    d_ff = _segsum_sc(g2_ff, dstp_ff, ones_tab)[0, :_N, 0]
    d_rt = _segsum_sc(g2_rt, dstp_rt, ones_tab)[0, :_N, 0]
    d_rb = _segsum_sc(g2_rb, dstp_rb, ones_tab)[0, :_N, 0]

    def prep(d):
        r = (1.0 / jnp.maximum(d, 1.0)).reshape(_N, 1)
        ind = (d > 0).astype(jnp.float32).reshape(_N, 1)
        return r, ind

    r_ff, i_ff = prep(d_ff)
    r_rt, i_rt = prep(d_rt)
    r_rb, i_rb = prep(d_rb)

    b2 = lambda b: b.reshape(1, -1)

    tu = feat_user.reshape(2 * _N, 128)
    ti = feat_item.reshape(2 * _N, 128)
    s_ff1 = _segsum_sc(g2_ff, dstp_ff, tu)
    s_rt1 = _segsum_sc(g2_rt, dstp_rt, tu)
    s_rb1 = _segsum_sc(g2_rb, dstp_rb, ti)
    hu1 = _combine_tc([([s_ff1], r_ff, i_ff, W1_ff, b2(b1_ff)),
                       ([s_rb1], r_rb, i_rb, W1_rb, b2(b1_rb))], 512, True)
    hi1 = _combine_tc([([s_rt1], r_rt, i_rt, W1_rt, b2(b1_rt))], 512, True)

    tu2 = hu1.reshape(4 * _N, 128)
    ti2 = hi1.reshape(4 * _N, 128)
    s_ff2a = _segsum_sc(_gidx(src_ff, 4, 0), dstp_ff, tu2)
    s_ff2b = _segsum_sc(_gidx(src_ff, 4, 2), dstp_ff, tu2)
    s_rt2a = _segsum_sc(_gidx(src_rt, 4, 0), dstp_rt, tu2)
    s_rt2b = _segsum_sc(_gidx(src_rt, 4, 2), dstp_rt, tu2)
    s_rb2a = _segsum_sc(_gidx(src_rb, 4, 0), dstp_rb, ti2)
    s_rb2b = _segsum_sc(_gidx(src_rb, 4, 2), dstp_rb, ti2)
    hu2 = _combine_tc([([s_ff2a, s_ff2b], r_ff, i_ff, W2_ff, b2(b2_ff)),
                       ([s_rb2a, s_rb2b], r_rb, i_rb, W2_rb, b2(b2_rb))], 512, True)
    hi2 = _combine_tc([([s_rt2a, s_rt2b], r_rt, i_rt, W2_rt, b2(b2_rt))], 512, True)

    p_ff = _linear_tc(hu2, W3_ff, b2(b3_ff))
    p_rb = _linear_tc(hi2, W3_rb, b2(b3_rb))
    s_ff3 = _segsum_sc(g2_ff, dstp_ff, p_ff.reshape(2 * _N, 128))
    s_rb3 = _segsum_sc(g2_rb, dstp_rb, p_rb.reshape(2 * _N, 128))
    return _final_tc(s_ff3, r_ff, s_rb3, r_rb)


# speedup vs baseline: 20.3559x; 1.1019x over previous
import functools

import jax
import jax.numpy as jnp
from jax import lax
from jax.experimental import pallas as pl
from jax.experimental.pallas import tpu as pltpu
from jax.experimental.pallas import tpu_sc as plsc

_N = 10000
_NP = 10240
_E = 160000
_K = 80
_NC = 2
_NS = 16
_BN = 2000


_NCH = (_E // _NS) // _K
_NCHP = 128


def _pad_chunks(a, lead):
    a3 = a.reshape(lead, _NCH, _K)
    return jnp.pad(a3, ((0, 0), (0, _NCHP - _NCH), (0, 0))).reshape(-1, _K)


def _segsum_sc(gidx, dstp, table):
    zrows = _NP // _NS
    mesh = plsc.VectorSubcoreMesh(core_axis_name="c", subcore_axis_name="s",
                                  num_cores=_NC, num_subcores=_NS)

    @functools.partial(
        pl.kernel,
        out_type=jax.ShapeDtypeStruct((2 * _NP, 128), jnp.float32),
        mesh=mesh,
        scratch_types=[
            pltpu.VMEM((8 * _K,), jnp.int32),
            pltpu.VMEM((_NCHP, _K), jnp.int32),
            pltpu.VMEM((2, _K, 128), jnp.float32),
            pltpu.VMEM_SHARED((_NP, 128), jnp.float32),
            pltpu.SemaphoreType.DMA,
            pltpu.SemaphoreType.DMA,
            pltpu.SemaphoreType.DMA,
        ],
    )
    def k(gidx_hbm, dst_hbm, table_hbm, out_hbm, idxg_v, dst_v, rows_v,
          acc_sh, gsem, ssem, isem):
        c = lax.axis_index("c")
        s = lax.axis_index("s")
        ebase = c * _E + s * (_E // _NS)

        pltpu.sync_copy(dst_hbm.at[pl.ds(s * _NCHP, _NCHP)], dst_v)

        @pl.loop(0, _K)
        def _(r):
            for j in range(8):
                rows_v[0, r, pl.ds(j * 16, 16)] = jnp.zeros((16,), jnp.float32)

        zb = s * zrows

        @pl.loop(0, zrows // _K)
        def _(t):
            pltpu.sync_copy(rows_v.at[0], acc_sh.at[pl.ds(zb + t * _K, _K)])

        plsc.subcore_barrier()

        def gather(bank, i, slot):
            return pltpu.async_copy(
                table_hbm.at[idxg_v.at[pl.ds(bank * 4 * _K + i * _K, _K)]],
                rows_v.at[slot], gsem)

        def scatter(g, slot):
            return pltpu.async_copy(rows_v.at[slot],
                                    acc_sh.at[dst_v.at[g]], ssem, add=True)

        def group(bank, base):
            g0 = gather(bank, 0, 0)
            g1 = gather(bank, 1, 1)
            g0.wait()
            s0 = scatter(base + 0, 0)
            g1.wait()
            s1 = scatter(base + 1, 1)
            s0.wait()
            g2 = gather(bank, 2, 0)
            s1.wait()
            g3 = gather(bank, 3, 1)
            g2.wait()
            s2 = scatter(base + 2, 0)
            g3.wait()
            s3 = scatter(base + 3, 1)
            s2.wait()
            s3.wait()

        def pf_start(bank, base):
            return pltpu.async_copy(
                gidx_hbm.at[pl.ds(ebase + base * _K, 4 * _K)],
                idxg_v.at[pl.ds(bank * 4 * _K, 4 * _K)], isem)

        def pf_wait(bank):
            pltpu.make_async_copy(
                gidx_hbm.at[pl.ds(ebase, 4 * _K)],
                idxg_v.at[pl.ds(bank * 4 * _K, 4 * _K)], isem).wait()

        pltpu.sync_copy(gidx_hbm.at[pl.ds(ebase, 4 * _K)],
                        idxg_v.at[pl.ds(0, 4 * _K)])

        @pl.loop(0, 15)
        def _(j):
            base_a = 8 * j
            pf_start(1, base_a + 4)
            group(0, base_a)
            pf_wait(1)
            pf_start(0, base_a + 8)
            group(1, base_a + 4)
            pf_wait(0)

        group(0, 120)

        pltpu.sync_copy(gidx_hbm.at[pl.ds(ebase + (_NCH - 1) * _K, _K)],
                        idxg_v.at[pl.ds(0, _K)])
        gather(0, 0, 0).wait()
        scatter(_NCH - 1, 0).wait()

        plsc.subcore_barrier()
        pltpu.sync_copy(acc_sh.at[pl.ds(zb, zrows)],
                        out_hbm.at[pl.ds(c * _NP + zb, zrows)])

    return k(gidx, dstp, table).reshape(2, _NP, 128)


def _combine_tc(terms, dout, leaky):
    args = []
    in_specs = []
    for halves, scale, ind, w, b2d in terms:
        for h in halves:
            args.append(h)
            in_specs.append(pl.BlockSpec((2, _BN, 128), lambda i: (0, i, 0)))
        args += [scale, ind, w, b2d]
        in_specs += [
            pl.BlockSpec((_BN, 1), lambda i: (i, 0)),
            pl.BlockSpec((_BN, 1), lambda i: (i, 0)),
            pl.BlockSpec(w.shape, lambda i: (0, 0)),
            pl.BlockSpec((1, dout), lambda i: (0, 0)),
        ]
    counts = [len(t[0]) for t in terms]

    def body(*refs):
        out_ref = refs[-1]
        acc = jnp.zeros((_BN, dout), jnp.float32)
        p = 0
        for cnt in counts:
            hrefs = refs[p:p + cnt]
            s_ref, i_ref, w_ref, b_ref = refs[p + cnt:p + cnt + 4]
            p += cnt + 4
            sc = s_ref[...]
            w = w_ref[...]
            for hi in range(cnt):
                x = hrefs[hi][...]
                for cc in range(2):
                    kb = hi * 2 + cc
                    acc += jnp.dot(x[cc] * sc, w[kb * 128:(kb + 1) * 128, :],
                                   preferred_element_type=jnp.float32)
            acc += i_ref[...] * b_ref[...]
        if leaky:
            acc = jnp.where(acc >= 0, acc, 0.01 * acc)
        out_ref[...] = acc

    return pl.pallas_call(
        body,
        grid=(_N // _BN,),
        in_specs=in_specs,
        out_specs=pl.BlockSpec((_BN, dout), lambda i: (i, 0)),
        out_shape=jax.ShapeDtypeStruct((_N, dout), jnp.float32),
    )(*args)


def _linear_tc(x, w, b2d):
    kdim = x.shape[1]
    dout = w.shape[1]

    def body(x_ref, w_ref, b_ref, out_ref):
        out_ref[...] = jnp.dot(x_ref[...], w_ref[...],
                               preferred_element_type=jnp.float32) + b_ref[...]

    return pl.pallas_call(
        body,
        grid=(_N // _BN,),
        in_specs=[
            pl.BlockSpec((_BN, kdim), lambda i: (i, 0)),
            pl.BlockSpec((kdim, dout), lambda i: (0, 0)),
            pl.BlockSpec((1, dout), lambda i: (0, 0)),
        ],
        out_specs=pl.BlockSpec((_BN, dout), lambda i: (i, 0)),
        out_shape=jax.ShapeDtypeStruct((_N, dout), jnp.float32),
    )(x, w, b2d)


def _final_tc(sff, rff, srb, rrb):
    def body(a_ref, ra_ref, b_ref, rb_ref, out_ref):
        a = a_ref[...]
        b = b_ref[...]
        ra = ra_ref[...]
        rb = rb_ref[...]
        out_ref[...] = jnp.concatenate(
            [a[0] * ra + b[0] * rb, a[1] * ra + b[1] * rb], axis=1)

    return pl.pallas_call(
        body,
        grid=(_N // _BN,),
        in_specs=[
            pl.BlockSpec((2, _BN, 128), lambda i: (0, i, 0)),
            pl.BlockSpec((_BN, 1), lambda i: (i, 0)),
            pl.BlockSpec((2, _BN, 128), lambda i: (0, i, 0)),
            pl.BlockSpec((_BN, 1), lambda i: (i, 0)),
        ],
        out_specs=pl.BlockSpec((_BN, 256), lambda i: (i, 0)),
        out_shape=jax.ShapeDtypeStruct((_N, 256), jnp.float32),
    )(sff, rff, srb, rrb)


def _gidx(src, m, o):
    base = m * src + o
    return jnp.concatenate([base, base + 1]).astype(jnp.int32)


def kernel(feat_user, feat_item, edge_ff, edge_rt, edge_rb,
           W1_ff, b1_ff, W1_rt, b1_rt, W1_rb, b1_rb,
           W2_ff, b2_ff, W2_rt, b2_rt, W2_rb, b2_rb,
           W3_ff, b3_ff, W3_rt, b3_rt, W3_rb, b3_rb):
    src_ff = edge_ff[0].astype(jnp.int32)
    dst_ff = edge_ff[1].astype(jnp.int32)
    src_rt = edge_rt[0].astype(jnp.int32)
    dst_rt = edge_rt[1].astype(jnp.int32)
    src_rb = edge_rb[0].astype(jnp.int32)
    dst_rb = edge_rb[1].astype(jnp.int32)

    dstp_ff = _pad_chunks(dst_ff, _NS)
    dstp_rt = _pad_chunks(dst_rt, _NS)
    dstp_rb = _pad_chunks(dst_rb, _NS)
    g2_ff = _gidx(src_ff, 2, 0)
    g2_rt = _gidx(src_rt, 2, 0)
    g2_rb = _gidx(src_rb, 2, 0)

    ones_tab = jnp.ones((2 * _N, 128), jnp.float32)
    d_ff = _segsum_sc(g2_ff, dstp_ff, ones_tab)[0, :_N, 0]
    d_rt = _segsum_sc(g2_rt, dstp_rt, ones_tab)[0, :_N, 0]
    d_rb = _segsum_sc(g2_rb, dstp_rb, ones_tab)[0, :_N, 0]

    def prep(d):
        r = (1.0 / jnp.maximum(d, 1.0)).reshape(_N, 1)
        ind = (d > 0).astype(jnp.float32).reshape(_N, 1)
        return r, ind

    r_ff, i_ff = prep(d_ff)
    r_rt, i_rt = prep(d_rt)
    r_rb, i_rb = prep(d_rb)

    b2 = lambda b: b.reshape(1, -1)

    tu = feat_user.reshape(2 * _N, 128)
    ti = feat_item.reshape(2 * _N, 128)
    s_ff1 = _segsum_sc(g2_ff, dstp_ff, tu)
    s_rt1 = _segsum_sc(g2_rt, dstp_rt, tu)
    s_rb1 = _segsum_sc(g2_rb, dstp_rb, ti)
    hu1 = _combine_tc([([s_ff1], r_ff, i_ff, W1_ff, b2(b1_ff)),
                       ([s_rb1], r_rb, i_rb, W1_rb, b2(b1_rb))], 512, True)
    hi1 = _combine_tc([([s_rt1], r_rt, i_rt, W1_rt, b2(b1_rt))], 512, True)

    tu2 = hu1.reshape(4 * _N, 128)
    ti2 = hi1.reshape(4 * _N, 128)
    s_ff2a = _segsum_sc(_gidx(src_ff, 4, 0), dstp_ff, tu2)
    s_ff2b = _segsum_sc(_gidx(src_ff, 4, 2), dstp_ff, tu2)
    s_rt2a = _segsum_sc(_gidx(src_rt, 4, 0), dstp_rt, tu2)
    s_rt2b = _segsum_sc(_gidx(src_rt, 4, 2), dstp_rt, tu2)
    s_rb2a = _segsum_sc(_gidx(src_rb, 4, 0), dstp_rb, ti2)
    s_rb2b = _segsum_sc(_gidx(src_rb, 4, 2), dstp_rb, ti2)
    hu2 = _combine_tc([([s_ff2a, s_ff2b], r_ff, i_ff, W2_ff, b2(b2_ff)),
                       ([s_rb2a, s_rb2b], r_rb, i_rb, W2_rb, b2(b2_rb))], 512, True)
    hi2 = _combine_tc([([s_rt2a, s_rt2b], r_rt, i_rt, W2_rt, b2(b2_rt))], 512, True)

    p_ff = _linear_tc(hu2, W3_ff, b2(b3_ff))
    p_rb = _linear_tc(hi2, W3_rb, b2(b3_rb))
    s_ff3 = _segsum_sc(g2_ff, dstp_ff, p_ff.reshape(2 * _N, 128))
    s_rb3 = _segsum_sc(g2_rb, dstp_rb, p_rb.reshape(2 * _N, 128))
    return _final_tc(s_ff3, r_ff, s_rb3, r_rb)


# speedup vs baseline: 22.4766x; 1.1042x over previous
import functools

import jax
import jax.numpy as jnp
from jax import lax
from jax.experimental import pallas as pl
from jax.experimental.pallas import tpu as pltpu
from jax.experimental.pallas import tpu_sc as plsc

_N = 10000
_NP = 10240
_E = 160000
_K = 80
_NC = 2
_NS = 16
_BN = 2000


_NCH = (_E // _NS) // _K
_NCHP = 128


def _pad_chunks(a, lead):
    a3 = a.reshape(lead, _NCH, _K)
    return jnp.pad(a3, ((0, 0), (0, _NCHP - _NCH), (0, 0))).reshape(-1, _K)


def _segsum_sc(gidx, dstp, table):
    zrows = _NP // _NS
    mesh = plsc.VectorSubcoreMesh(core_axis_name="c", subcore_axis_name="s",
                                  num_cores=_NC, num_subcores=_NS)

    @functools.partial(
        pl.kernel,
        out_type=jax.ShapeDtypeStruct((2 * _NP, 128), jnp.float32),
        mesh=mesh,
        scratch_types=[
            pltpu.VMEM((8 * _K,), jnp.int32),
            pltpu.VMEM((_NCHP, _K), jnp.int32),
            pltpu.VMEM((2, _K, 128), jnp.float32),
            pltpu.VMEM_SHARED((_NP, 128), jnp.float32),
            pltpu.SemaphoreType.DMA,
            pltpu.SemaphoreType.DMA,
            pltpu.SemaphoreType.DMA,
        ],
    )
    def k(gidx_hbm, dst_hbm, table_hbm, out_hbm, idxg_v, dst_v, rows_v,
          acc_sh, gsem, ssem, isem):
        c = lax.axis_index("c")
        s = lax.axis_index("s")
        ebase = c * _E + s * (_E // _NS)

        pltpu.sync_copy(dst_hbm.at[pl.ds(s * _NCHP, _NCHP)], dst_v)

        @pl.loop(0, _K)
        def _(r):
            for j in range(8):
                rows_v[0, r, pl.ds(j * 16, 16)] = jnp.zeros((16,), jnp.float32)

        zb = s * zrows

        @pl.loop(0, zrows // _K)
        def _(t):
            pltpu.sync_copy(rows_v.at[0], acc_sh.at[pl.ds(zb + t * _K, _K)])

        plsc.subcore_barrier()

        def gather(bank, i, slot):
            return pltpu.async_copy(
                table_hbm.at[idxg_v.at[pl.ds(bank * 4 * _K + i * _K, _K)]],
                rows_v.at[slot], gsem)

        def scatter(g, slot):
            return pltpu.async_copy(rows_v.at[slot],
                                    acc_sh.at[dst_v.at[g]], ssem, add=True)

        def group(bank, base):
            g0 = gather(bank, 0, 0)
            g1 = gather(bank, 1, 1)
            g0.wait()
            s0 = scatter(base + 0, 0)
            g1.wait()
            s1 = scatter(base + 1, 1)
            s0.wait()
            g2 = gather(bank, 2, 0)
            s1.wait()
            g3 = gather(bank, 3, 1)
            g2.wait()
            s2 = scatter(base + 2, 0)
            g3.wait()
            s3 = scatter(base + 3, 1)
            s2.wait()
            s3.wait()

        def pf_start(bank, base):
            return pltpu.async_copy(
                gidx_hbm.at[pl.ds(ebase + base * _K, 4 * _K)],
                idxg_v.at[pl.ds(bank * 4 * _K, 4 * _K)], isem)

        def pf_wait(bank):
            pltpu.make_async_copy(
                gidx_hbm.at[pl.ds(ebase, 4 * _K)],
                idxg_v.at[pl.ds(bank * 4 * _K, 4 * _K)], isem).wait()

        pltpu.sync_copy(gidx_hbm.at[pl.ds(ebase, 4 * _K)],
                        idxg_v.at[pl.ds(0, 4 * _K)])

        @pl.loop(0, 15)
        def _(j):
            base_a = 8 * j
            pf_start(1, base_a + 4)
            group(0, base_a)
            pf_wait(1)
            pf_start(0, base_a + 8)
            group(1, base_a + 4)
            pf_wait(0)

        group(0, 120)

        pltpu.sync_copy(gidx_hbm.at[pl.ds(ebase + (_NCH - 1) * _K, _K)],
                        idxg_v.at[pl.ds(0, _K)])
        gather(0, 0, 0).wait()
        scatter(_NCH - 1, 0).wait()

        plsc.subcore_barrier()
        pltpu.sync_copy(acc_sh.at[pl.ds(zb, zrows)],
                        out_hbm.at[pl.ds(c * _NP + zb, zrows)])

    return k(gidx, dstp, table).reshape(2, _NP, 128)


def _degrees_sc(dstp_ff, dstp_rt, dstp_rb):
    zrows = _NP // _NS
    mesh = plsc.VectorSubcoreMesh(core_axis_name="c", subcore_axis_name="s",
                                  num_cores=_NC, num_subcores=_NS)

    @functools.partial(
        pl.kernel,
        out_type=jax.ShapeDtypeStruct((2 * 3 * _NP, 128), jnp.float32),
        mesh=mesh,
        scratch_types=[
            pltpu.VMEM((_K, 128), jnp.float32),
            pltpu.VMEM((_K, 128), jnp.float32),
            pltpu.VMEM((_NCHP, _K), jnp.int32),
            pltpu.VMEM_SHARED((_NP, 128), jnp.float32),
            pltpu.SemaphoreType.DMA,
        ],
    )
    def k(dff_hbm, drt_hbm, drb_hbm, out_hbm, ones_v, zbuf_v, dst_v,
          acc_sh, ssem):
        c = lax.axis_index("c")
        s = lax.axis_index("s")
        zb = s * zrows

        @pl.loop(0, _K)
        def _(r):
            for j in range(8):
                ones_v[r, pl.ds(j * 16, 16)] = jnp.ones((16,), jnp.float32)
                zbuf_v[r, pl.ds(j * 16, 16)] = jnp.zeros((16,), jnp.float32)

        for r, dref in enumerate([dff_hbm, drt_hbm, drb_hbm]):
            pltpu.sync_copy(dref.at[pl.ds(s * _NCHP, _NCHP)], dst_v)

            @pl.loop(0, zrows // _K)
            def _(t):
                pltpu.sync_copy(zbuf_v, acc_sh.at[pl.ds(zb + t * _K, _K)])

            plsc.subcore_barrier()

            @pl.loop(0, 15)
            def _(j):
                sds = [pltpu.async_copy(ones_v,
                                        acc_sh.at[dst_v.at[8 * j + b]],
                                        ssem, add=True) for b in range(8)]
                for d in sds:
                    d.wait()

            sds = [pltpu.async_copy(ones_v, acc_sh.at[dst_v.at[120 + b]],
                                    ssem, add=True) for b in range(5)]
            for d in sds:
                d.wait()

            plsc.subcore_barrier()
            pltpu.sync_copy(acc_sh.at[pl.ds(zb, zrows)],
                            out_hbm.at[pl.ds((c * 3 + r) * _NP + zb, zrows)])

    return k(dstp_ff, dstp_rt, dstp_rb)


def _combine_tc(terms, dout, leaky):
    args = []
    in_specs = []
    for halves, scale, ind, w, b2d in terms:
        for h in halves:
            args.append(h)
            in_specs.append(pl.BlockSpec((2, _BN, 128), lambda i: (0, i, 0)))
        args += [scale, ind, w, b2d]
        in_specs += [
            pl.BlockSpec((_BN, 1), lambda i: (i, 0)),
            pl.BlockSpec((_BN, 1), lambda i: (i, 0)),
            pl.BlockSpec(w.shape, lambda i: (0, 0)),
            pl.BlockSpec((1, dout), lambda i: (0, 0)),
        ]
    counts = [len(t[0]) for t in terms]

    def body(*refs):
        out_ref = refs[-1]
        acc = jnp.zeros((_BN, dout), jnp.float32)
        p = 0
        for cnt in counts:
            hrefs = refs[p:p + cnt]
            s_ref, i_ref, w_ref, b_ref = refs[p + cnt:p + cnt + 4]
            p += cnt + 4
            sc = s_ref[...]
            w = w_ref[...]
            for hi in range(cnt):
                x = hrefs[hi][...]
                for cc in range(2):
                    kb = hi * 2 + cc
                    acc += jnp.dot(x[cc] * sc, w[kb * 128:(kb + 1) * 128, :],
                                   preferred_element_type=jnp.float32)
            acc += i_ref[...] * b_ref[...]
        if leaky:
            acc = jnp.where(acc >= 0, acc, 0.01 * acc)
        out_ref[...] = acc

    return pl.pallas_call(
        body,
        grid=(_N // _BN,),
        in_specs=in_specs,
        out_specs=pl.BlockSpec((_BN, dout), lambda i: (i, 0)),
        out_shape=jax.ShapeDtypeStruct((_N, dout), jnp.float32),
    )(*args)


def _linear_tc(x, w, b2d):
    kdim = x.shape[1]
    dout = w.shape[1]

    def body(x_ref, w_ref, b_ref, out_ref):
        out_ref[...] = jnp.dot(x_ref[...], w_ref[...],
                               preferred_element_type=jnp.float32) + b_ref[...]

    return pl.pallas_call(
        body,
        grid=(_N // _BN,),
        in_specs=[
            pl.BlockSpec((_BN, kdim), lambda i: (i, 0)),
            pl.BlockSpec((kdim, dout), lambda i: (0, 0)),
            pl.BlockSpec((1, dout), lambda i: (0, 0)),
        ],
        out_specs=pl.BlockSpec((_BN, dout), lambda i: (i, 0)),
        out_shape=jax.ShapeDtypeStruct((_N, dout), jnp.float32),
    )(x, w, b2d)


def _final_tc(sff, rff, srb, rrb):
    def body(a_ref, ra_ref, b_ref, rb_ref, out_ref):
        a = a_ref[...]
        b = b_ref[...]
        ra = ra_ref[...]
        rb = rb_ref[...]
        out_ref[...] = jnp.concatenate(
            [a[0] * ra + b[0] * rb, a[1] * ra + b[1] * rb], axis=1)

    return pl.pallas_call(
        body,
        grid=(_N // _BN,),
        in_specs=[
            pl.BlockSpec((2, _BN, 128), lambda i: (0, i, 0)),
            pl.BlockSpec((_BN, 1), lambda i: (i, 0)),
            pl.BlockSpec((2, _BN, 128), lambda i: (0, i, 0)),
            pl.BlockSpec((_BN, 1), lambda i: (i, 0)),
        ],
        out_specs=pl.BlockSpec((_BN, 256), lambda i: (i, 0)),
        out_shape=jax.ShapeDtypeStruct((_N, 256), jnp.float32),
    )(sff, rff, srb, rrb)


def _gidx(src, m, o):
    base = m * src + o
    return jnp.concatenate([base, base + 1]).astype(jnp.int32)


def kernel(feat_user, feat_item, edge_ff, edge_rt, edge_rb,
           W1_ff, b1_ff, W1_rt, b1_rt, W1_rb, b1_rb,
           W2_ff, b2_ff, W2_rt, b2_rt, W2_rb, b2_rb,
           W3_ff, b3_ff, W3_rt, b3_rt, W3_rb, b3_rb):
    src_ff = edge_ff[0].astype(jnp.int32)
    dst_ff = edge_ff[1].astype(jnp.int32)
    src_rt = edge_rt[0].astype(jnp.int32)
    dst_rt = edge_rt[1].astype(jnp.int32)
    src_rb = edge_rb[0].astype(jnp.int32)
    dst_rb = edge_rb[1].astype(jnp.int32)

    dstp_ff = _pad_chunks(dst_ff, _NS)
    dstp_rt = _pad_chunks(dst_rt, _NS)
    dstp_rb = _pad_chunks(dst_rb, _NS)
    g2_ff = _gidx(src_ff, 2, 0)
    g2_rt = _gidx(src_rt, 2, 0)
    g2_rb = _gidx(src_rb, 2, 0)

    degs = _degrees_sc(dstp_ff, dstp_rt, dstp_rb)
    d_ff = degs[0 * _NP:0 * _NP + _N, 0]
    d_rt = degs[1 * _NP:1 * _NP + _N, 0]
    d_rb = degs[2 * _NP:2 * _NP + _N, 0]

    def prep(d):
        r = (1.0 / jnp.maximum(d, 1.0)).reshape(_N, 1)
        ind = (d > 0).astype(jnp.float32).reshape(_N, 1)
        return r, ind

    r_ff, i_ff = prep(d_ff)
    r_rt, i_rt = prep(d_rt)
    r_rb, i_rb = prep(d_rb)

    b2 = lambda b: b.reshape(1, -1)

    tu = feat_user.reshape(2 * _N, 128)
    ti = feat_item.reshape(2 * _N, 128)
    s_ff1 = _segsum_sc(g2_ff, dstp_ff, tu)
    s_rt1 = _segsum_sc(g2_rt, dstp_rt, tu)
    s_rb1 = _segsum_sc(g2_rb, dstp_rb, ti)
    hu1 = _combine_tc([([s_ff1], r_ff, i_ff, W1_ff, b2(b1_ff)),
                       ([s_rb1], r_rb, i_rb, W1_rb, b2(b1_rb))], 512, True)
    hi1 = _combine_tc([([s_rt1], r_rt, i_rt, W1_rt, b2(b1_rt))], 512, True)

    tu2 = hu1.reshape(4 * _N, 128)
    ti2 = hi1.reshape(4 * _N, 128)
    s_ff2a = _segsum_sc(_gidx(src_ff, 4, 0), dstp_ff, tu2)
    s_ff2b = _segsum_sc(_gidx(src_ff, 4, 2), dstp_ff, tu2)
    s_rt2a = _segsum_sc(_gidx(src_rt, 4, 0), dstp_rt, tu2)
    s_rt2b = _segsum_sc(_gidx(src_rt, 4, 2), dstp_rt, tu2)
    s_rb2a = _segsum_sc(_gidx(src_rb, 4, 0), dstp_rb, ti2)
    s_rb2b = _segsum_sc(_gidx(src_rb, 4, 2), dstp_rb, ti2)
    hu2 = _combine_tc([([s_ff2a, s_ff2b], r_ff, i_ff, W2_ff, b2(b2_ff)),
                       ([s_rb2a, s_rb2b], r_rb, i_rb, W2_rb, b2(b2_rb))], 512, True)
    hi2 = _combine_tc([([s_rt2a, s_rt2b], r_rt, i_rt, W2_rt, b2(b2_rt))], 512, True)

    p_ff = _linear_tc(hu2, W3_ff, b2(b3_ff))
    p_rb = _linear_tc(hi2, W3_rb, b2(b3_rb))
    s_ff3 = _segsum_sc(g2_ff, dstp_ff, p_ff.reshape(2 * _N, 128))
    s_rb3 = _segsum_sc(g2_rb, dstp_rb, p_rb.reshape(2 * _N, 128))
    return _final_tc(s_ff3, r_ff, s_rb3, r_rb)
